# baseline (speedup 1.0000x reference)
import functools

import jax
import jax.numpy as jnp
from jax.experimental import pallas as pl
from jax.experimental.pallas import tpu as pltpu

N = 1_000_000
L = 16
H = 2
BT = 65536
GRID = (N + BT - 1) // BT


def _tc_body(w_ref, x_ref, o_ref):
    x = x_ref[...]
    w = w_ref[...]
    o_ref[...] = jax.lax.dot_general(
        w, x, (((1,), (0,)), ((), ())),
        preferred_element_type=jnp.float32,
    )


_tc_run = pl.pallas_call(
    _tc_body,
    grid=(GRID,),
    in_specs=[
        pl.BlockSpec((H, L), lambda i: (0, 0)),
        pl.BlockSpec((L, BT), lambda i: (0, i)),
    ],
    out_specs=pl.BlockSpec((H, BT), lambda i: (0, i)),
    out_shape=jax.ShapeDtypeStruct((H, N), jnp.float32),
)


def kernel(x, atomic_energy):
    out_t = _tc_run(atomic_energy, x.T)
    return out_t.T


# speedup vs baseline: 1.0531x; 1.0531x over previous
import functools

import jax
import jax.numpy as jnp
from jax.experimental import pallas as pl
from jax.experimental.pallas import tpu as pltpu

N = 1_000_000
L = 16
H = 2
BT = 262144
GRID = (N + BT - 1) // BT


def _tc_body(w_ref, x_ref, o_ref):
    x = x_ref[...]
    w = w_ref[...]
    o_ref[...] = jax.lax.dot_general(
        w, x, (((1,), (0,)), ((), ())),
        preferred_element_type=jnp.float32,
    )


_tc_run = pl.pallas_call(
    _tc_body,
    grid=(GRID,),
    in_specs=[
        pl.BlockSpec((H, L), lambda i: (0, 0)),
        pl.BlockSpec((L, BT), lambda i: (0, i)),
    ],
    out_specs=pl.BlockSpec((H, BT), lambda i: (0, i)),
    out_shape=jax.ShapeDtypeStruct((H, N), jnp.float32),
)


def kernel(x, atomic_energy):
    out_t = _tc_run(atomic_energy, x.T)
    return out_t.T
